# baseline (speedup 1.0000x reference)
import functools

import jax
import jax.numpy as jnp
from jax.experimental import pallas as pl

T_TILE = 1024


def _router_body(x_ref, w1_ref, b1_ref, w2_ref, b2_ref, wg_ref,
                 logits_ref, probs_ref, tki_ref, tkp_ref, feat_ref):
    x = x_ref[...]
    h = jnp.maximum(
        jnp.dot(x, w1_ref[...], preferred_element_type=jnp.float32)
        + b1_ref[...], 0.0)
    h = jnp.maximum(
        jnp.dot(h, w2_ref[...], preferred_element_type=jnp.float32)
        + b2_ref[...], 0.0)
    feat_ref[...] = h
    logits = jnp.dot(h, wg_ref[...], preferred_element_type=jnp.float32)
    logits_ref[...] = logits

    m = jnp.max(logits, axis=1, keepdims=True)
    e = jnp.exp(logits - m)
    s = jnp.sum(e, axis=1, keepdims=True)
    probs = e / s
    probs_ref[...] = probs

    t, n_e = probs.shape
    iota = jax.lax.broadcasted_iota(jnp.int32, (t, n_e), 1)
    p1 = jnp.max(probs, axis=1, keepdims=True)
    i1 = jnp.min(jnp.where(probs == p1, iota, n_e), axis=1, keepdims=True)
    masked = jnp.where(iota == i1, -1.0, probs)
    p2 = jnp.max(masked, axis=1, keepdims=True)
    i2 = jnp.min(jnp.where(masked == p2, iota, n_e), axis=1, keepdims=True)
    tki_ref[...] = jnp.concatenate([i1, i2], axis=1)
    denom = p1 + p2
    tkp_ref[...] = jnp.concatenate([p1 / denom, p2 / denom], axis=1)


@jax.jit
def kernel(x, W1, b1, W2, b2, Wg):
    ntok, hidden = x.shape
    rhid = W1.shape[1]
    n_e = Wg.shape[1]
    grid = ntok // T_TILE

    out_shapes = (
        jax.ShapeDtypeStruct((ntok, n_e), jnp.float32),
        jax.ShapeDtypeStruct((ntok, n_e), jnp.float32),
        jax.ShapeDtypeStruct((ntok, 2), jnp.int32),
        jax.ShapeDtypeStruct((ntok, 2), jnp.float32),
        jax.ShapeDtypeStruct((ntok, rhid), jnp.float32),
    )
    tok_spec = lambda w: pl.BlockSpec((T_TILE, w), lambda i: (i, 0))
    fixed_spec = lambda a, b: pl.BlockSpec((a, b), lambda i: (0, 0))

    return pl.pallas_call(
        _router_body,
        grid=(grid,),
        in_specs=[
            tok_spec(hidden),
            fixed_spec(hidden, rhid),
            fixed_spec(1, rhid),
            fixed_spec(rhid, rhid),
            fixed_spec(1, rhid),
            fixed_spec(rhid, n_e),
        ],
        out_specs=(
            tok_spec(n_e),
            tok_spec(n_e),
            tok_spec(2),
            tok_spec(2),
            tok_spec(rhid),
        ),
        out_shape=out_shapes,
    )(x, W1, b1.reshape(1, -1), W2, b2.reshape(1, -1), Wg)
